# Initial kernel scaffold; baseline (speedup 1.0000x reference)
import functools

import jax
import jax.numpy as jnp
from jax import lax
from jax.experimental import pallas as pl
from jax.experimental.pallas import tpu as pltpu
from jax.experimental.pallas import tpu_sc as plsc

N = 10000
E = 320000
D = 128
R = 64

NC = 2
NS = 16
NW = NC * NS

EPT = E // NW
CK = 80
NCHUNK = EPT // CK
RPT = N // NS
ZR = 125


def _prep_body(h_ref, norm_ref, r_ref, hn_ref, negr_ref):
    hn_ref[...] = h_ref[...] * norm_ref[...]
    negr_ref[...] = -r_ref[...]


def _prep(h, norm, r):
    return pl.pallas_call(
        _prep_body,
        out_shape=[
            jax.ShapeDtypeStruct((N, D), jnp.float32),
            jax.ShapeDtypeStruct((R, D), jnp.float32),
        ],
    )(h, norm, r)


def _sc_agg_body(hn_hbm, negr_hbm, src_hbm, dst_hbm, rel_hbm, out_hbm,
                 a_sp, hrows, rrows, src_v, dst_v, rel_v, zbuf, sem1, sem2):
    cid = lax.axis_index("c")
    sid = lax.axis_index("s")
    wid = sid * NC + cid

    def zb(k, carry):
        i = k // (D // 16)
        j = k - i * (D // 16)
        zbuf[i, pl.ds(j * 16, 16)] = jnp.zeros((16,), jnp.float32)
        return carry

    lax.fori_loop(0, ZR * (D // 16), zb, 0)

    def zinit(p, carry):
        pltpu.sync_copy(zbuf, a_sp.at[pl.ds(sid * RPT + p * ZR, ZR)])
        return carry

    lax.fori_loop(0, RPT // ZR, zinit, 0)
    plsc.subcore_barrier()

    base = wid * EPT

    def chunk(i, carry):
        off = base + i * CK
        pltpu.sync_copy(src_hbm.at[pl.ds(off, CK)], src_v)
        pltpu.sync_copy(rel_hbm.at[pl.ds(off, CK)], rel_v)
        pltpu.sync_copy(dst_hbm.at[pl.ds(off, CK)], dst_v)
        g1 = pltpu.async_copy(hn_hbm.at[src_v], hrows, sem1)
        g2 = pltpu.async_copy(negr_hbm.at[rel_v], rrows, sem2)
        g1.wait()
        g2.wait()
        pltpu.sync_copy(hrows, a_sp.at[dst_v], add=True)
        pltpu.sync_copy(rrows, a_sp.at[dst_v], add=True)
        return carry

    lax.fori_loop(0, NCHUNK, chunk, 0)
    plsc.subcore_barrier()

    pltpu.sync_copy(a_sp.at[pl.ds(sid * RPT, RPT)],
                    out_hbm.at[cid, pl.ds(sid * RPT, RPT)])


def _sc_agg(hn, negr, src, dst, rel):
    mesh = plsc.VectorSubcoreMesh(core_axis_name="c", subcore_axis_name="s",
                                  num_cores=NC, num_subcores=NS)
    f = functools.partial(
        pl.kernel,
        out_type=jax.ShapeDtypeStruct((NC, N, D), jnp.float32),
        mesh=mesh,
        scratch_types=[
            pltpu.VMEM_SHARED((N, D), jnp.float32),
            pltpu.VMEM((CK, D), jnp.float32),
            pltpu.VMEM((CK, D), jnp.float32),
            pltpu.VMEM((CK,), jnp.int32),
            pltpu.VMEM((CK,), jnp.int32),
            pltpu.VMEM((CK,), jnp.int32),
            pltpu.VMEM((ZR, D), jnp.float32),
            pltpu.SemaphoreType.DMA,
            pltpu.SemaphoreType.DMA,
        ],
    )(_sc_agg_body)
    return f(hn, negr, src, dst, rel)


def _dense_body(hn_ref, a_ref, norm_ref, w_ref, wm_ref, b_ref, out_ref):
    hn = hn_ref[...]
    agg = (a_ref[0] + a_ref[1]) * norm_ref[...]
    t = jnp.dot(hn, w_ref[...], preferred_element_type=jnp.float32)
    t = t + jnp.dot(agg, wm_ref[...], preferred_element_type=jnp.float32)
    out_ref[...] = jnp.maximum(t + b_ref[...], 0.0)


def _dense(hn, a, norm, w, wm, b2):
    blk = 1000
    grid = N // blk
    return pl.pallas_call(
        _dense_body,
        grid=(grid,),
        in_specs=[
            pl.BlockSpec((blk, D), lambda i: (i, 0)),
            pl.BlockSpec((NC, blk, D), lambda i: (0, i, 0)),
            pl.BlockSpec((blk, 1), lambda i: (i, 0)),
            pl.BlockSpec((D, D), lambda i: (0, 0)),
            pl.BlockSpec((D, D), lambda i: (0, 0)),
            pl.BlockSpec((1, D), lambda i: (0, 0)),
        ],
        out_specs=pl.BlockSpec((blk, D), lambda i: (i, 0)),
        out_shape=jax.ShapeDtypeStruct((N, D), jnp.float32),
    )(hn, a, norm, w, wm, b2)


def kernel(h, r, edge_index, edge_rel, norm, W_msg, W, b):
    src = edge_index[0]
    dst = edge_index[1]
    hn, negr = _prep(h, norm, r)
    a = _sc_agg(hn, negr, src, dst, edge_rel)
    return _dense(hn, a, norm, W, W_msg, b.reshape(1, D))


# speedup vs baseline: 5.2661x; 5.2661x over previous
import functools

import jax
import jax.numpy as jnp
from jax import lax
from jax.experimental import pallas as pl
from jax.experimental.pallas import tpu as pltpu
from jax.experimental.pallas import tpu_sc as plsc

N = 10000
E = 320000
D = 128
R = 64

NC = 2
NS = 16
NW = NC * NS

EPT = E // NW
CK = 80
NCHUNK = EPT // CK
RPT = 624
TAIL = N - NS * RPT
ZR = 104


def _prep_body(h_ref, norm_ref, r_ref, hn_ref, negr_ref):
    hn_ref[...] = h_ref[...] * norm_ref[...]
    negr_ref[...] = -r_ref[...]


def _prep(h, norm, r):
    return pl.pallas_call(
        _prep_body,
        out_shape=[
            jax.ShapeDtypeStruct((N, D), jnp.float32),
            jax.ShapeDtypeStruct((R, D), jnp.float32),
        ],
    )(h, norm, r)


def _sc_agg_body(hn_hbm, negr_hbm, src_hbm, dst_hbm, rel_hbm, out_hbm,
                 a_sp, hrows, rrows, src_v, dst_v, rel_v, zbuf, sem1, sem2):
    cid = lax.axis_index("c")
    sid = lax.axis_index("s")
    wid = sid * NC + cid

    def zb(k, carry):
        i = k // (D // 16)
        j = k - i * (D // 16)
        zbuf[i, pl.ds(j * 16, 16)] = jnp.zeros((16,), jnp.float32)
        return carry

    lax.fori_loop(0, ZR * (D // 16), zb, 0)

    def zinit(p, carry):
        pltpu.sync_copy(zbuf, a_sp.at[pl.ds(sid * RPT + p * ZR, ZR)])
        return carry

    lax.fori_loop(0, RPT // ZR, zinit, 0)

    @pl.when(sid == NS - 1)
    def _():
        pltpu.sync_copy(zbuf.at[pl.ds(0, TAIL)], a_sp.at[pl.ds(NS * RPT, TAIL)])

    plsc.subcore_barrier()

    base = wid * EPT

    def chunk(i, carry):
        off = base + i * CK
        pltpu.sync_copy(src_hbm.at[pl.ds(off, CK)], src_v)
        pltpu.sync_copy(rel_hbm.at[pl.ds(off, CK)], rel_v)
        pltpu.sync_copy(dst_hbm.at[pl.ds(off, CK)], dst_v)
        g1 = pltpu.async_copy(hn_hbm.at[src_v], hrows, sem1)
        g2 = pltpu.async_copy(negr_hbm.at[rel_v], rrows, sem2)
        g1.wait()
        g2.wait()
        pltpu.sync_copy(hrows, a_sp.at[dst_v], add=True)
        pltpu.sync_copy(rrows, a_sp.at[dst_v], add=True)
        return carry

    lax.fori_loop(0, NCHUNK, chunk, 0)
    plsc.subcore_barrier()

    pltpu.sync_copy(a_sp.at[pl.ds(sid * RPT, RPT)],
                    out_hbm.at[cid, pl.ds(sid * RPT, RPT)])

    @pl.when(sid == NS - 1)
    def _():
        pltpu.sync_copy(a_sp.at[pl.ds(NS * RPT, TAIL)],
                        out_hbm.at[cid, pl.ds(NS * RPT, TAIL)])


def _sc_agg(hn, negr, src, dst, rel):
    mesh = plsc.VectorSubcoreMesh(core_axis_name="c", subcore_axis_name="s",
                                  num_cores=NC, num_subcores=NS)
    f = functools.partial(
        pl.kernel,
        out_type=jax.ShapeDtypeStruct((NC, N, D), jnp.float32),
        mesh=mesh,
        scratch_types=[
            pltpu.VMEM_SHARED((N, D), jnp.float32),
            pltpu.VMEM((CK, D), jnp.float32),
            pltpu.VMEM((CK, D), jnp.float32),
            pltpu.VMEM((CK,), jnp.int32),
            pltpu.VMEM((CK,), jnp.int32),
            pltpu.VMEM((CK,), jnp.int32),
            pltpu.VMEM((ZR, D), jnp.float32),
            pltpu.SemaphoreType.DMA,
            pltpu.SemaphoreType.DMA,
        ],
    )(_sc_agg_body)
    return f(hn, negr, src, dst, rel)


def _dense_body(hn_ref, a_ref, norm_ref, w_ref, wm_ref, b_ref, out_ref):
    hn = hn_ref[...]
    agg = (a_ref[0] + a_ref[1]) * norm_ref[...]
    t = jnp.dot(hn, w_ref[...], preferred_element_type=jnp.float32)
    t = t + jnp.dot(agg, wm_ref[...], preferred_element_type=jnp.float32)
    out_ref[...] = jnp.maximum(t + b_ref[...], 0.0)


def _dense(hn, a, norm, w, wm, b2):
    blk = 1000
    grid = N // blk
    return pl.pallas_call(
        _dense_body,
        grid=(grid,),
        in_specs=[
            pl.BlockSpec((blk, D), lambda i: (i, 0)),
            pl.BlockSpec((NC, blk, D), lambda i: (0, i, 0)),
            pl.BlockSpec((blk, 1), lambda i: (i, 0)),
            pl.BlockSpec((D, D), lambda i: (0, 0)),
            pl.BlockSpec((D, D), lambda i: (0, 0)),
            pl.BlockSpec((1, D), lambda i: (0, 0)),
        ],
        out_specs=pl.BlockSpec((blk, D), lambda i: (i, 0)),
        out_shape=jax.ShapeDtypeStruct((N, D), jnp.float32),
    )(hn, a, norm, w, wm, b2)


def kernel(h, r, edge_index, edge_rel, norm, W_msg, W, b):
    src = edge_index[0]
    dst = edge_index[1]
    hn, negr = _prep(h, norm, r)
    a = _sc_agg(hn, negr, src, dst, edge_rel)
    return _dense(hn, a, norm, W, W_msg, b.reshape(1, D))


# speedup vs baseline: 7.7402x; 1.4698x over previous
import functools

import jax
import jax.numpy as jnp
from jax import lax
from jax.experimental import pallas as pl
from jax.experimental.pallas import tpu as pltpu
from jax.experimental.pallas import tpu_sc as plsc

N = 10000
E = 320000
D = 128
R = 64

NC = 2
NS = 16
NW = NC * NS

EPT = E // NW
CK = 80
NCHUNK = EPT // CK
RPT = 624
TAIL = N - NS * RPT
ZR = 16


def _prep_body(h_ref, norm_ref, r_ref, hn_ref, negr_ref):
    hn_ref[...] = h_ref[...] * norm_ref[...]
    negr_ref[...] = -r_ref[...]


def _prep(h, norm, r):
    return pl.pallas_call(
        _prep_body,
        out_shape=[
            jax.ShapeDtypeStruct((N, D), jnp.float32),
            jax.ShapeDtypeStruct((R, D), jnp.float32),
        ],
    )(h, norm, r)


def _sc_agg_body(hn_hbm, negr_hbm, src_hbm, dst_hbm, rel_hbm, out_hbm,
                 a_sp, hrows0, rrows0, hrows1, rrows1,
                 srcb0, dstb0, relb0, srcb1, dstb1, relb1, zbuf,
                 si0, si1, sg0, sg1):
    cid = lax.axis_index("c")
    sid = lax.axis_index("s")
    wid = sid * NC + cid
    base = wid * EPT

    idxb = ((srcb0, dstb0, relb0, si0), (srcb1, dstb1, relb1, si1))
    rowb = ((hrows0, rrows0, sg0), (hrows1, rrows1, sg1))

    def issue_idx(i, b):
        src_v, dst_v, rel_v, sem = idxb[b]
        off = base + i * CK
        pltpu.async_copy(src_hbm.at[pl.ds(off, CK)], src_v, sem)
        pltpu.async_copy(dst_hbm.at[pl.ds(off, CK)], dst_v, sem)
        pltpu.async_copy(rel_hbm.at[pl.ds(off, CK)], rel_v, sem)

    def wait_idx(b):
        src_v, dst_v, rel_v, sem = idxb[b]
        pltpu.make_async_copy(src_hbm.at[pl.ds(base, CK)], src_v, sem).wait()
        pltpu.make_async_copy(dst_hbm.at[pl.ds(base, CK)], dst_v, sem).wait()
        pltpu.make_async_copy(rel_hbm.at[pl.ds(base, CK)], rel_v, sem).wait()

    def issue_g(b):
        src_v, dst_v, rel_v, _ = idxb[b]
        hbuf, rbuf, sem = rowb[b]
        pltpu.async_copy(hn_hbm.at[src_v], hbuf, sem)
        pltpu.async_copy(negr_hbm.at[rel_v], rbuf, sem)

    def wait_g(b):
        src_v, dst_v, rel_v, _ = idxb[b]
        hbuf, rbuf, sem = rowb[b]
        pltpu.make_async_copy(hn_hbm.at[src_v], hbuf, sem).wait()
        pltpu.make_async_copy(negr_hbm.at[rel_v], rbuf, sem).wait()

    def scat(b):
        src_v, dst_v, rel_v, _ = idxb[b]
        hbuf, rbuf, _ = rowb[b]
        pltpu.sync_copy(hbuf, a_sp.at[dst_v], add=True)
        pltpu.sync_copy(rbuf, a_sp.at[dst_v], add=True)

    issue_idx(0, 0)
    issue_idx(1, 1)

    def zb(k, carry):
        i = k // (D // 16)
        j = k - i * (D // 16)
        zbuf[i, pl.ds(j * 16, 16)] = jnp.zeros((16,), jnp.float32)
        return carry

    lax.fori_loop(0, ZR * (D // 16), zb, 0)

    def zinit(p, carry):
        pltpu.sync_copy(zbuf, a_sp.at[pl.ds(sid * RPT + p * ZR, ZR)])
        return carry

    lax.fori_loop(0, RPT // ZR, zinit, 0)

    @pl.when(sid == NS - 1)
    def _():
        pltpu.sync_copy(zbuf.at[pl.ds(0, TAIL)], a_sp.at[pl.ds(NS * RPT, TAIL)])

    plsc.subcore_barrier()

    wait_idx(0)
    issue_g(0)

    def pair(p, carry):
        i0 = 2 * p
        wait_g(0)
        wait_idx(1)
        issue_g(1)
        scat(0)
        issue_idx(i0 + 2, 0)
        wait_g(1)
        wait_idx(0)
        issue_g(0)
        scat(1)

        @pl.when(i0 + 3 < NCHUNK)
        def _():
            issue_idx(i0 + 3, 1)

        return carry

    lax.fori_loop(0, (NCHUNK - 1) // 2, pair, 0)
    wait_g(0)
    scat(0)
    plsc.subcore_barrier()

    pltpu.sync_copy(a_sp.at[pl.ds(sid * RPT, RPT)],
                    out_hbm.at[cid, pl.ds(sid * RPT, RPT)])

    @pl.when(sid == NS - 1)
    def _():
        pltpu.sync_copy(a_sp.at[pl.ds(NS * RPT, TAIL)],
                        out_hbm.at[cid, pl.ds(NS * RPT, TAIL)])


def _sc_agg(hn, negr, src, dst, rel):
    mesh = plsc.VectorSubcoreMesh(core_axis_name="c", subcore_axis_name="s",
                                  num_cores=NC, num_subcores=NS)
    f = functools.partial(
        pl.kernel,
        out_type=jax.ShapeDtypeStruct((NC, N, D), jnp.float32),
        mesh=mesh,
        scratch_types=[
            pltpu.VMEM_SHARED((N, D), jnp.float32),
            pltpu.VMEM((CK, D), jnp.float32),
            pltpu.VMEM((CK, D), jnp.float32),
            pltpu.VMEM((CK, D), jnp.float32),
            pltpu.VMEM((CK, D), jnp.float32),
            pltpu.VMEM((CK,), jnp.int32),
            pltpu.VMEM((CK,), jnp.int32),
            pltpu.VMEM((CK,), jnp.int32),
            pltpu.VMEM((CK,), jnp.int32),
            pltpu.VMEM((CK,), jnp.int32),
            pltpu.VMEM((CK,), jnp.int32),
            pltpu.VMEM((ZR, D), jnp.float32),
            pltpu.SemaphoreType.DMA,
            pltpu.SemaphoreType.DMA,
            pltpu.SemaphoreType.DMA,
            pltpu.SemaphoreType.DMA,
        ],
    )(_sc_agg_body)
    return f(hn, negr, src, dst, rel)


def _dense_body(hn_ref, a_ref, norm_ref, w_ref, wm_ref, b_ref, out_ref):
    hn = hn_ref[...]
    agg = (a_ref[0] + a_ref[1]) * norm_ref[...]
    t = jnp.dot(hn, w_ref[...], preferred_element_type=jnp.float32)
    t = t + jnp.dot(agg, wm_ref[...], preferred_element_type=jnp.float32)
    out_ref[...] = jnp.maximum(t + b_ref[...], 0.0)


def _dense(hn, a, norm, w, wm, b2):
    blk = 1000
    grid = N // blk
    return pl.pallas_call(
        _dense_body,
        grid=(grid,),
        in_specs=[
            pl.BlockSpec((blk, D), lambda i: (i, 0)),
            pl.BlockSpec((NC, blk, D), lambda i: (0, i, 0)),
            pl.BlockSpec((blk, 1), lambda i: (i, 0)),
            pl.BlockSpec((D, D), lambda i: (0, 0)),
            pl.BlockSpec((D, D), lambda i: (0, 0)),
            pl.BlockSpec((1, D), lambda i: (0, 0)),
        ],
        out_specs=pl.BlockSpec((blk, D), lambda i: (i, 0)),
        out_shape=jax.ShapeDtypeStruct((N, D), jnp.float32),
    )(hn, a, norm, w, wm, b2)


def kernel(h, r, edge_index, edge_rel, norm, W_msg, W, b):
    src = edge_index[0]
    dst = edge_index[1]
    hn, negr = _prep(h, norm, r)
    a = _sc_agg(hn, negr, src, dst, edge_rel)
    return _dense(hn, a, norm, W, W_msg, b.reshape(1, D))


# speedup vs baseline: 8.2955x; 1.0717x over previous
import functools

import jax
import jax.numpy as jnp
from jax import lax
from jax.experimental import pallas as pl
from jax.experimental.pallas import tpu as pltpu
from jax.experimental.pallas import tpu_sc as plsc

N = 10000
E = 320000
D = 128
R = 64
H = D // 2

NC = 2
NS = 16
L = 16

EPT = E // NS
CK = 80
NCHUNK = EPT // CK

RPT = 624
TAIL = N - NS * RPT
NR = N * R
CSPAN = 39936
CTAIL = NR - NS * CSPAN
ZC = 4992


def _prep_body(h_ref, norm_ref, hn2_ref):
    hn = h_ref[...] * norm_ref[...]
    hn2_ref[0] = hn[:, :H]
    hn2_ref[1] = hn[:, H:]


def _prep(h, norm):
    return pl.pallas_call(
        _prep_body,
        out_shape=jax.ShapeDtypeStruct((NC, N, H), jnp.float32),
    )(h, norm)


def _sc_agg_body(hn_hbm, ep_hbm, a_out, c_out,
                 a_sp, c_sp,
                 idxb0, idxb1, so0, so1, db0, db1, fb0, fb1,
                 hrow0, hrow1, ones, zbuf, zc,
                 si0, si1, sg0, sg1, ss0, ss1, sq0, sq1):
    cid = lax.axis_index("c")
    sid = lax.axis_index("s")

    idxb = (idxb0, idxb1)
    sidx = (si0, si1)
    so = (so0, so1)
    db = (db0, db1)
    fb = (fb0, fb1)
    hrow = (hrow0, hrow1)
    sg = (sg0, sg1)
    ss = (ss0, ss1)
    sq = (sq0, sq1)
    srcbase = cid * N

    def issue_idx(i, b):
        pltpu.async_copy(ep_hbm.at[sid, i], idxb[b], sidx[b])

    def wait_idx(b):
        pltpu.make_async_copy(ep_hbm.at[sid, 0], idxb[b], sidx[b]).wait()

    def vec_phase(b):
        for q in range(CK // L):
            sl = pl.ds(q * L, L)
            sv = idxb[b][0, sl]
            dv = idxb[b][1, sl]
            rv = idxb[b][2, sl]
            so[b][sl] = sv + srcbase
            db[b][sl] = dv
            fb[b][sl] = dv * R + rv

    def issue_g(b):
        pltpu.async_copy(hn_hbm.at[so[b]], hrow[b], sg[b])

    def wait_g(b):
        pltpu.make_async_copy(hn_hbm.at[so[b]], hrow[b], sg[b]).wait()

    def issue_scat(b):
        pltpu.async_copy(hrow[b], a_sp.at[db[b]], ss[b], add=True)
        pltpu.async_copy(ones, c_sp.at[fb[b]], sq[b], add=True)

    def wait_scat(b):
        pltpu.make_async_copy(hrow[b], a_sp.at[db[b]], ss[b]).wait()
        pltpu.make_async_copy(ones, c_sp.at[fb[b]], sq[b]).wait()

    issue_idx(0, 0)
    issue_idx(1, 1)

    for q in range(CK // L):
        ones[pl.ds(q * L, L)] = jnp.ones((L,), jnp.float32)

    def zb_fill(k, carry):
        i = k // (H // L)
        j = k - i * (H // L)
        zbuf[i, pl.ds(j * L, L)] = jnp.zeros((L,), jnp.float32)
        return carry

    lax.fori_loop(0, 16 * (H // L), zb_fill, 0)

    def zc_fill(k, carry):
        zc[pl.ds(k * L, L)] = jnp.zeros((L,), jnp.float32)
        return carry

    lax.fori_loop(0, ZC // L, zc_fill, 0)

    def zinit_a(p, carry):
        pltpu.sync_copy(zbuf, a_sp.at[pl.ds(sid * RPT + p * 16, 16)])
        return carry

    lax.fori_loop(0, RPT // 16, zinit_a, 0)

    def zinit_c(p, carry):
        pltpu.sync_copy(zc, c_sp.at[pl.ds(sid * CSPAN + p * ZC, ZC)])
        return carry

    lax.fori_loop(0, CSPAN // ZC, zinit_c, 0)

    @pl.when(sid == NS - 1)
    def _():
        pltpu.sync_copy(zbuf, a_sp.at[pl.ds(NS * RPT, TAIL)])
        pltpu.sync_copy(zc.at[pl.ds(0, CTAIL)], c_sp.at[pl.ds(NS * CSPAN, CTAIL)])

    plsc.subcore_barrier()

    wait_idx(0)
    vec_phase(0)
    issue_g(0)
    issue_idx(2, 0)

    def half(i, b):
        wait_g(b)
        issue_scat(b)
        nb = 1 - b

        @pl.when(i + 1 < NCHUNK)
        def _():
            wait_idx(nb)

            @pl.when(i >= 1)
            def _():
                wait_scat(nb)

            vec_phase(nb)
            issue_g(nb)

            @pl.when(i + 3 < NCHUNK)
            def _():
                issue_idx(i + 3, nb)

    def pair(p, carry):
        half(2 * p, 0)
        half(2 * p + 1, 1)
        return carry

    lax.fori_loop(0, NCHUNK // 2, pair, 0)
    wait_scat(0)
    wait_scat(1)
    plsc.subcore_barrier()

    pltpu.sync_copy(a_sp.at[pl.ds(sid * RPT, RPT)],
                    a_out.at[cid, pl.ds(sid * RPT, RPT)])
    pltpu.sync_copy(c_sp.at[pl.ds(sid * CSPAN, CSPAN)],
                    c_out.at[cid, pl.ds(sid * CSPAN, CSPAN)])

    @pl.when(sid == NS - 1)
    def _():
        pltpu.sync_copy(a_sp.at[pl.ds(NS * RPT, TAIL)],
                        a_out.at[cid, pl.ds(NS * RPT, TAIL)])
        pltpu.sync_copy(c_sp.at[pl.ds(NS * CSPAN, CTAIL)],
                        c_out.at[cid, pl.ds(NS * CSPAN, CTAIL)])


def _sc_agg(hn2flat, epack):
    mesh = plsc.VectorSubcoreMesh(core_axis_name="c", subcore_axis_name="s",
                                  num_cores=NC, num_subcores=NS)
    f = functools.partial(
        pl.kernel,
        out_type=[
            jax.ShapeDtypeStruct((NC, N, H), jnp.float32),
            jax.ShapeDtypeStruct((NC, NR), jnp.float32),
        ],
        mesh=mesh,
        scratch_types=[
            pltpu.VMEM_SHARED((N, H), jnp.float32),
            pltpu.VMEM_SHARED((NR,), jnp.float32),
            pltpu.VMEM((3, CK), jnp.int32),
            pltpu.VMEM((3, CK), jnp.int32),
            pltpu.VMEM((CK,), jnp.int32),
            pltpu.VMEM((CK,), jnp.int32),
            pltpu.VMEM((CK,), jnp.int32),
            pltpu.VMEM((CK,), jnp.int32),
            pltpu.VMEM((CK,), jnp.int32),
            pltpu.VMEM((CK,), jnp.int32),
            pltpu.VMEM((CK, H), jnp.float32),
            pltpu.VMEM((CK, H), jnp.float32),
            pltpu.VMEM((CK,), jnp.float32),
            pltpu.VMEM((16, H), jnp.float32),
            pltpu.VMEM((ZC,), jnp.float32),
            pltpu.SemaphoreType.DMA,
            pltpu.SemaphoreType.DMA,
            pltpu.SemaphoreType.DMA,
            pltpu.SemaphoreType.DMA,
            pltpu.SemaphoreType.DMA,
            pltpu.SemaphoreType.DMA,
            pltpu.SemaphoreType.DMA,
            pltpu.SemaphoreType.DMA,
        ],
        compiler_params=pltpu.CompilerParams(use_tc_tiling_on_sc=False),
    )(_sc_agg_body)
    return f(hn2flat, epack)


def _dense_body(hn2_ref, a_ref, c2_ref, norm_ref, r_ref, w_ref, wm_ref,
                b_ref, out_ref):
    hn = jnp.concatenate([hn2_ref[0], hn2_ref[1]], axis=1)
    sh = jnp.concatenate([a_ref[0], a_ref[1]], axis=1)
    csum = (c2_ref[0] + c2_ref[1]) * 0.5
    srel = jnp.dot(csum, r_ref[...], preferred_element_type=jnp.float32)
    agg = (sh - srel) * norm_ref[...]
    t = jnp.dot(hn, w_ref[...], preferred_element_type=jnp.float32)
    t = t + jnp.dot(agg, wm_ref[...], preferred_element_type=jnp.float32)
    out_ref[...] = jnp.maximum(t + b_ref[...], 0.0)


def _dense(hn2, a, c2, norm, r, w, wm, b2):
    blk = 1000
    grid = N // blk
    return pl.pallas_call(
        _dense_body,
        grid=(grid,),
        in_specs=[
            pl.BlockSpec((NC, blk, H), lambda i: (0, i, 0)),
            pl.BlockSpec((NC, blk, H), lambda i: (0, i, 0)),
            pl.BlockSpec((NC, blk, R), lambda i: (0, i, 0)),
            pl.BlockSpec((blk, 1), lambda i: (i, 0)),
            pl.BlockSpec((R, D), lambda i: (0, 0)),
            pl.BlockSpec((D, D), lambda i: (0, 0)),
            pl.BlockSpec((D, D), lambda i: (0, 0)),
            pl.BlockSpec((1, D), lambda i: (0, 0)),
        ],
        out_specs=pl.BlockSpec((blk, D), lambda i: (i, 0)),
        out_shape=jax.ShapeDtypeStruct((N, D), jnp.float32),
    )(hn2, a, c2, norm, r, w, wm, b2)


def kernel(h, r, edge_index, edge_rel, norm, W_msg, W, b):
    src = edge_index[0]
    dst = edge_index[1]
    epack = (jnp.stack([src, dst, edge_rel], axis=0)
             .reshape(3, NS, NCHUNK, CK).transpose(1, 2, 0, 3))
    hn2 = _prep(h, norm)
    a, c = _sc_agg(hn2.reshape(NC * N, H), epack)
    return _dense(hn2, a, c.reshape(NC, N, R), norm, r, W, W_msg,
                  b.reshape(1, D))


# speedup vs baseline: 11.2360x; 1.3545x over previous
import functools

import jax
import jax.numpy as jnp
from jax import lax
from jax.experimental import pallas as pl
from jax.experimental.pallas import tpu as pltpu
from jax.experimental.pallas import tpu_sc as plsc

N = 10000
E = 320000
D = 128
R = 64
H = D // 2

NC = 2
NS = 16
L = 16

EPT = E // NS
CK = 80
NCHUNK = EPT // CK

RPT = 624
TAIL = N - NS * RPT
NR = N * R
CSPAN = 39936
CTAIL = NR - NS * CSPAN
ZC = 4992


def _prep_body(h_ref, norm_ref, hn2_ref):
    hn = h_ref[...] * norm_ref[...]
    hn2_ref[0] = hn[:, :H]
    hn2_ref[1] = hn[:, H:]


def _prep(h, norm):
    return pl.pallas_call(
        _prep_body,
        out_shape=jax.ShapeDtypeStruct((NC, N, H), jnp.float32),
    )(h, norm)


NB = 4
GD = 2


def _sc_agg_body(hn_hbm, ep_hbm, a_out, c_out,
                 a_sp, c_sp,
                 idxb0, idxb1, idxb2, idxb3,
                 so0, so1, so2, so3, db0, db1, db2, db3,
                 fb0, fb1, fb2, fb3,
                 hrow0, hrow1, hrow2, hrow3, ones, zbuf, zc,
                 si0, si1, si2, si3, sg0, sg1, sg2, sg3,
                 ss0, ss1, ss2, ss3, sq0, sq1, sq2, sq3):
    cid = lax.axis_index("c")
    sid = lax.axis_index("s")

    idxb = (idxb0, idxb1, idxb2, idxb3)
    sidx = (si0, si1, si2, si3)
    so = (so0, so1, so2, so3)
    db = (db0, db1, db2, db3)
    fb = (fb0, fb1, fb2, fb3)
    hrow = (hrow0, hrow1, hrow2, hrow3)
    sg = (sg0, sg1, sg2, sg3)
    ss = (ss0, ss1, ss2, ss3)
    sq = (sq0, sq1, sq2, sq3)
    srcbase = cid * N

    def issue_idx(i, b):
        pltpu.async_copy(ep_hbm.at[sid, i], idxb[b], sidx[b])

    def wait_idx(b):
        pltpu.make_async_copy(ep_hbm.at[sid, 0], idxb[b], sidx[b]).wait()

    def vec_phase(b):
        for q in range(CK // L):
            sl = pl.ds(q * L, L)
            sv = idxb[b][0, sl]
            dv = idxb[b][1, sl]
            rv = idxb[b][2, sl]
            so[b][sl] = sv + srcbase
            db[b][sl] = dv
            fb[b][sl] = dv * R + rv

    def issue_g(b):
        pltpu.async_copy(hn_hbm.at[so[b]], hrow[b], sg[b])

    def wait_g(b):
        pltpu.make_async_copy(hn_hbm.at[so[b]], hrow[b], sg[b]).wait()

    def issue_scat(b):
        pltpu.async_copy(hrow[b], a_sp.at[db[b]], ss[b], add=True)
        pltpu.async_copy(ones, c_sp.at[fb[b]], sq[b], add=True)

    def wait_scat(b):
        pltpu.make_async_copy(hrow[b], a_sp.at[db[b]], ss[b]).wait()
        pltpu.make_async_copy(ones, c_sp.at[fb[b]], sq[b]).wait()

    for k in range(NB):
        issue_idx(k, k)

    for q in range(CK // L):
        ones[pl.ds(q * L, L)] = jnp.ones((L,), jnp.float32)

    def zb_fill(k, carry):
        i = k // (H // L)
        j = k - i * (H // L)
        zbuf[i, pl.ds(j * L, L)] = jnp.zeros((L,), jnp.float32)
        return carry

    lax.fori_loop(0, 16 * (H // L), zb_fill, 0)

    def zc_fill(k, carry):
        zc[pl.ds(k * L, L)] = jnp.zeros((L,), jnp.float32)
        return carry

    lax.fori_loop(0, ZC // L, zc_fill, 0)

    def zinit_a(p, carry):
        pltpu.sync_copy(zbuf, a_sp.at[pl.ds(sid * RPT + p * 16, 16)])
        return carry

    lax.fori_loop(0, RPT // 16, zinit_a, 0)

    def zinit_c(p, carry):
        pltpu.sync_copy(zc, c_sp.at[pl.ds(sid * CSPAN + p * ZC, ZC)])
        return carry

    lax.fori_loop(0, CSPAN // ZC, zinit_c, 0)

    @pl.when(sid == NS - 1)
    def _():
        pltpu.sync_copy(zbuf, a_sp.at[pl.ds(NS * RPT, TAIL)])
        pltpu.sync_copy(zc.at[pl.ds(0, CTAIL)], c_sp.at[pl.ds(NS * CSPAN, CTAIL)])

    plsc.subcore_barrier()

    for k in range(GD):
        wait_idx(k)
        vec_phase(k)
        issue_g(k)
        issue_idx(k + NB, k)

    def retire(i, b):
        wait_g(b)
        issue_scat(b)
        j = i + GD
        bj = (b + GD) % NB

        @pl.when(j < NCHUNK)
        def _():
            wait_idx(bj)

            @pl.when(j >= NB)
            def _():
                wait_scat(bj)

            vec_phase(bj)
            issue_g(bj)

            @pl.when(j + NB < NCHUNK)
            def _():
                issue_idx(j + NB, bj)

    def quad(p, carry):
        for u in range(NB):
            i = NB * p + u

            @pl.when(i < NCHUNK)
            def _():
                retire(i, u)

        return carry

    lax.fori_loop(0, (NCHUNK + NB - 1) // NB, quad, 0)
    for b in range(NB):
        wait_scat(b)
    plsc.subcore_barrier()

    pltpu.sync_copy(a_sp.at[pl.ds(sid * RPT, RPT)],
                    a_out.at[cid, pl.ds(sid * RPT, RPT)])
    pltpu.sync_copy(c_sp.at[pl.ds(sid * CSPAN, CSPAN)],
                    c_out.at[cid, pl.ds(sid * CSPAN, CSPAN)])

    @pl.when(sid == NS - 1)
    def _():
        pltpu.sync_copy(a_sp.at[pl.ds(NS * RPT, TAIL)],
                        a_out.at[cid, pl.ds(NS * RPT, TAIL)])
        pltpu.sync_copy(c_sp.at[pl.ds(NS * CSPAN, CTAIL)],
                        c_out.at[cid, pl.ds(NS * CSPAN, CTAIL)])


def _sc_agg(hn2flat, epack):
    mesh = plsc.VectorSubcoreMesh(core_axis_name="c", subcore_axis_name="s",
                                  num_cores=NC, num_subcores=NS)
    f = functools.partial(
        pl.kernel,
        out_type=[
            jax.ShapeDtypeStruct((NC, N, H), jnp.float32),
            jax.ShapeDtypeStruct((NC, NR), jnp.float32),
        ],
        mesh=mesh,
        scratch_types=(
            [pltpu.VMEM_SHARED((N, H), jnp.float32),
             pltpu.VMEM_SHARED((NR,), jnp.float32)]
            + [pltpu.VMEM((3, CK), jnp.int32) for _ in range(NB)]
            + [pltpu.VMEM((CK,), jnp.int32) for _ in range(3 * NB)]
            + [pltpu.VMEM((CK, H), jnp.float32) for _ in range(NB)]
            + [pltpu.VMEM((CK,), jnp.float32),
               pltpu.VMEM((16, H), jnp.float32),
               pltpu.VMEM((ZC,), jnp.float32)]
            + [pltpu.SemaphoreType.DMA for _ in range(4 * NB)]
        ),
        compiler_params=pltpu.CompilerParams(use_tc_tiling_on_sc=False),
    )(_sc_agg_body)
    return f(hn2flat, epack)


def _dense_body(hn2_ref, a_ref, c2_ref, norm_ref, r_ref, w_ref, wm_ref,
                b_ref, out_ref):
    hn = jnp.concatenate([hn2_ref[0], hn2_ref[1]], axis=1)
    sh = jnp.concatenate([a_ref[0], a_ref[1]], axis=1)
    csum = (c2_ref[0] + c2_ref[1]) * 0.5
    srel = jnp.dot(csum, r_ref[...], preferred_element_type=jnp.float32)
    agg = (sh - srel) * norm_ref[...]
    t = jnp.dot(hn, w_ref[...], preferred_element_type=jnp.float32)
    t = t + jnp.dot(agg, wm_ref[...], preferred_element_type=jnp.float32)
    out_ref[...] = jnp.maximum(t + b_ref[...], 0.0)


def _dense(hn2, a, c2, norm, r, w, wm, b2):
    blk = 1000
    grid = N // blk
    return pl.pallas_call(
        _dense_body,
        grid=(grid,),
        in_specs=[
            pl.BlockSpec((NC, blk, H), lambda i: (0, i, 0)),
            pl.BlockSpec((NC, blk, H), lambda i: (0, i, 0)),
            pl.BlockSpec((NC, blk, R), lambda i: (0, i, 0)),
            pl.BlockSpec((blk, 1), lambda i: (i, 0)),
            pl.BlockSpec((R, D), lambda i: (0, 0)),
            pl.BlockSpec((D, D), lambda i: (0, 0)),
            pl.BlockSpec((D, D), lambda i: (0, 0)),
            pl.BlockSpec((1, D), lambda i: (0, 0)),
        ],
        out_specs=pl.BlockSpec((blk, D), lambda i: (i, 0)),
        out_shape=jax.ShapeDtypeStruct((N, D), jnp.float32),
    )(hn2, a, c2, norm, r, w, wm, b2)


def kernel(h, r, edge_index, edge_rel, norm, W_msg, W, b):
    src = edge_index[0]
    dst = edge_index[1]
    epack = (jnp.stack([src, dst, edge_rel], axis=0)
             .reshape(3, NS, NCHUNK, CK).transpose(1, 2, 0, 3))
    hn2 = _prep(h, norm)
    a, c = _sc_agg(hn2.reshape(NC * N, H), epack)
    return _dense(hn2, a, c.reshape(NC, N, R), norm, r, W, W_msg,
                  b.reshape(1, D))


# speedup vs baseline: 12.3802x; 1.1018x over previous
import functools

import jax
import jax.numpy as jnp
from jax import lax
from jax.experimental import pallas as pl
from jax.experimental.pallas import tpu as pltpu
from jax.experimental.pallas import tpu_sc as plsc

N = 10000
E = 320000
D = 128
R = 64
H = D // 2

NC = 2
NS = 16
L = 16

EPT = E // NS
CK = 80
NCHUNK = EPT // CK

RPT = 624
TAIL = N - NS * RPT
NR = N * R
CSPAN = 39936
CTAIL = NR - NS * CSPAN
ZC = 4992
ZB = 104


def _prep_body(h_ref, norm_ref, hn2_ref):
    hn = h_ref[...] * norm_ref[...]
    hn2_ref[0] = hn[:, :H]
    hn2_ref[1] = hn[:, H:]


def _prep(h, norm):
    return pl.pallas_call(
        _prep_body,
        out_shape=jax.ShapeDtypeStruct((NC, N, H), jnp.float32),
    )(h, norm)


NB = 6
GD = 3


def _sc_agg_body(hn_hbm, ep_hbm, a_out, c_out,
                 a_sp, c_sp,
                 idxb0, idxb1, idxb2, idxb3, idxb4, idxb5,
                 so0, so1, so2, so3, so4, so5,
                 db0, db1, db2, db3, db4, db5,
                 fb0, fb1, fb2, fb3, fb4, fb5,
                 hrow0, hrow1, hrow2, hrow3, hrow4, hrow5,
                 ones, zbuf, zc,
                 si0, si1, si2, si3, si4, si5,
                 sg0, sg1, sg2, sg3, sg4, sg5,
                 ss0, ss1, ss2, ss3, ss4, ss5,
                 sq0, sq1, sq2, sq3, sq4, sq5, sz):
    cid = lax.axis_index("c")
    sid = lax.axis_index("s")

    idxb = (idxb0, idxb1, idxb2, idxb3, idxb4, idxb5)
    sidx = (si0, si1, si2, si3, si4, si5)
    so = (so0, so1, so2, so3, so4, so5)
    db = (db0, db1, db2, db3, db4, db5)
    fb = (fb0, fb1, fb2, fb3, fb4, fb5)
    hrow = (hrow0, hrow1, hrow2, hrow3, hrow4, hrow5)
    sg = (sg0, sg1, sg2, sg3, sg4, sg5)
    ss = (ss0, ss1, ss2, ss3, ss4, ss5)
    sq = (sq0, sq1, sq2, sq3, sq4, sq5)
    srcbase = cid * N

    def issue_idx(i, b):
        pltpu.async_copy(ep_hbm.at[sid, i], idxb[b], sidx[b])

    def wait_idx(b):
        pltpu.make_async_copy(ep_hbm.at[sid, 0], idxb[b], sidx[b]).wait()

    def vec_phase(b):
        for q in range(CK // L):
            sl = pl.ds(q * L, L)
            sv = idxb[b][0, sl]
            dv = idxb[b][1, sl]
            rv = idxb[b][2, sl]
            so[b][sl] = sv + srcbase
            db[b][sl] = dv
            fb[b][sl] = dv * R + rv

    def issue_g(b):
        pltpu.async_copy(hn_hbm.at[so[b]], hrow[b], sg[b])

    def wait_g(b):
        pltpu.make_async_copy(hn_hbm.at[so[b]], hrow[b], sg[b]).wait()

    def issue_scat(b):
        pltpu.async_copy(hrow[b], a_sp.at[db[b]], ss[b], add=True)

        @pl.when(cid == b % 2)
        def _():
            pltpu.async_copy(ones, c_sp.at[fb[b]], sq[b], add=True)

    def wait_scat(b):
        pltpu.make_async_copy(hrow[b], a_sp.at[db[b]], ss[b]).wait()

        @pl.when(cid == b % 2)
        def _():
            pltpu.make_async_copy(ones, c_sp.at[fb[b]], sq[b]).wait()

    for k in range(NB):
        issue_idx(k, k)

    for q in range(CK // L):
        ones[pl.ds(q * L, L)] = jnp.ones((L,), jnp.float32)

    def zb_fill(k, carry):
        i = k // (H // L)
        j = k - i * (H // L)
        zbuf[i, pl.ds(j * L, L)] = jnp.zeros((L,), jnp.float32)
        return carry

    lax.fori_loop(0, ZB * (H // L), zb_fill, 0)

    def zc_fill(k, carry):
        zc[pl.ds(k * L, L)] = jnp.zeros((L,), jnp.float32)
        return carry

    lax.fori_loop(0, ZC // L, zc_fill, 0)

    def zinit_a(p, carry):
        pltpu.async_copy(zbuf, a_sp.at[pl.ds(sid * RPT + p * ZB, ZB)], sz)
        return carry

    lax.fori_loop(0, RPT // ZB, zinit_a, 0)

    def zinit_c(p, carry):
        pltpu.async_copy(zc, c_sp.at[pl.ds(sid * CSPAN + p * ZC, ZC)], sz)
        return carry

    lax.fori_loop(0, CSPAN // ZC, zinit_c, 0)

    @pl.when(sid == NS - 1)
    def _():
        pltpu.async_copy(zbuf.at[pl.ds(0, TAIL)],
                         a_sp.at[pl.ds(NS * RPT, TAIL)], sz)
        pltpu.async_copy(zc.at[pl.ds(0, CTAIL)],
                         c_sp.at[pl.ds(NS * CSPAN, CTAIL)], sz)

    for k in range(GD):
        wait_idx(k)
        vec_phase(k)
        issue_g(k)
        issue_idx(k + NB, k)

    def zwait_a(p, carry):
        pltpu.make_async_copy(zbuf, a_sp.at[pl.ds(sid * RPT + p * ZB, ZB)],
                              sz).wait()
        return carry

    lax.fori_loop(0, RPT // ZB, zwait_a, 0)

    def zwait_c(p, carry):
        pltpu.make_async_copy(zc, c_sp.at[pl.ds(sid * CSPAN + p * ZC, ZC)],
                              sz).wait()
        return carry

    lax.fori_loop(0, CSPAN // ZC, zwait_c, 0)

    @pl.when(sid == NS - 1)
    def _():
        pltpu.make_async_copy(zbuf.at[pl.ds(0, TAIL)],
                              a_sp.at[pl.ds(NS * RPT, TAIL)], sz).wait()
        pltpu.make_async_copy(zc.at[pl.ds(0, CTAIL)],
                              c_sp.at[pl.ds(NS * CSPAN, CTAIL)], sz).wait()

    plsc.subcore_barrier()

    def retire(i, b):
        wait_g(b)
        issue_scat(b)
        j = i + GD
        bj = (b + GD) % NB

        @pl.when(j < NCHUNK)
        def _():
            wait_idx(bj)

            @pl.when(j >= NB)
            def _():
                wait_scat(bj)

            vec_phase(bj)
            issue_g(bj)

            @pl.when(j + NB < NCHUNK)
            def _():
                issue_idx(j + NB, bj)

    def quad(p, carry):
        for u in range(NB):
            i = NB * p + u

            @pl.when(i < NCHUNK)
            def _():
                retire(i, u)

        return carry

    lax.fori_loop(0, (NCHUNK + NB - 1) // NB, quad, 0)
    for b in range(NB):
        wait_scat(b)
    plsc.subcore_barrier()

    pltpu.sync_copy(a_sp.at[pl.ds(sid * RPT, RPT)],
                    a_out.at[cid, pl.ds(sid * RPT, RPT)])
    pltpu.sync_copy(c_sp.at[pl.ds(sid * CSPAN, CSPAN)],
                    c_out.at[cid, pl.ds(sid * CSPAN, CSPAN)])

    @pl.when(sid == NS - 1)
    def _():
        pltpu.sync_copy(a_sp.at[pl.ds(NS * RPT, TAIL)],
                        a_out.at[cid, pl.ds(NS * RPT, TAIL)])
        pltpu.sync_copy(c_sp.at[pl.ds(NS * CSPAN, CTAIL)],
                        c_out.at[cid, pl.ds(NS * CSPAN, CTAIL)])


def _sc_agg(hn2flat, epack):
    mesh = plsc.VectorSubcoreMesh(core_axis_name="c", subcore_axis_name="s",
                                  num_cores=NC, num_subcores=NS)
    f = functools.partial(
        pl.kernel,
        out_type=[
            jax.ShapeDtypeStruct((NC, N, H), jnp.float32),
            jax.ShapeDtypeStruct((NC, NR), jnp.float32),
        ],
        mesh=mesh,
        scratch_types=(
            [pltpu.VMEM_SHARED((N, H), jnp.float32),
             pltpu.VMEM_SHARED((NR,), jnp.float32)]
            + [pltpu.VMEM((3, CK), jnp.int32) for _ in range(NB)]
            + [pltpu.VMEM((CK,), jnp.int32) for _ in range(3 * NB)]
            + [pltpu.VMEM((CK, H), jnp.float32) for _ in range(NB)]
            + [pltpu.VMEM((CK,), jnp.float32),
               pltpu.VMEM((ZB, H), jnp.float32),
               pltpu.VMEM((ZC,), jnp.float32)]
            + [pltpu.SemaphoreType.DMA for _ in range(4 * NB + 1)]
        ),
        compiler_params=pltpu.CompilerParams(use_tc_tiling_on_sc=False),
    )(_sc_agg_body)
    return f(hn2flat, epack)


def _dense_body(hn2_ref, a_ref, c2_ref, norm_ref, r_ref, w_ref, wm_ref,
                b_ref, out_ref):
    hn = jnp.concatenate([hn2_ref[0], hn2_ref[1]], axis=1)
    sh = jnp.concatenate([a_ref[0], a_ref[1]], axis=1)
    csum = c2_ref[0] + c2_ref[1]
    srel = jnp.dot(csum, r_ref[...], preferred_element_type=jnp.float32)
    agg = (sh - srel) * norm_ref[...]
    t = jnp.dot(hn, w_ref[...], preferred_element_type=jnp.float32)
    t = t + jnp.dot(agg, wm_ref[...], preferred_element_type=jnp.float32)
    out_ref[...] = jnp.maximum(t + b_ref[...], 0.0)


def _dense(hn2, a, c2, norm, r, w, wm, b2):
    blk = 1000
    grid = N // blk
    return pl.pallas_call(
        _dense_body,
        grid=(grid,),
        in_specs=[
            pl.BlockSpec((NC, blk, H), lambda i: (0, i, 0)),
            pl.BlockSpec((NC, blk, H), lambda i: (0, i, 0)),
            pl.BlockSpec((NC, blk, R), lambda i: (0, i, 0)),
            pl.BlockSpec((blk, 1), lambda i: (i, 0)),
            pl.BlockSpec((R, D), lambda i: (0, 0)),
            pl.BlockSpec((D, D), lambda i: (0, 0)),
            pl.BlockSpec((D, D), lambda i: (0, 0)),
            pl.BlockSpec((1, D), lambda i: (0, 0)),
        ],
        out_specs=pl.BlockSpec((blk, D), lambda i: (i, 0)),
        out_shape=jax.ShapeDtypeStruct((N, D), jnp.float32),
    )(hn2, a, c2, norm, r, w, wm, b2)


def kernel(h, r, edge_index, edge_rel, norm, W_msg, W, b):
    src = edge_index[0]
    dst = edge_index[1]
    epack = (jnp.stack([src, dst, edge_rel], axis=0)
             .reshape(3, NS, NCHUNK, CK).transpose(1, 2, 0, 3))
    hn2 = _prep(h, norm)
    a, c = _sc_agg(hn2.reshape(NC * N, H), epack)
    return _dense(hn2, a, c.reshape(NC, N, R), norm, r, W, W_msg,
                  b.reshape(1, D))
